# baseline (speedup 1.0000x reference)
import functools

import jax
import jax.numpy as jnp
from jax import lax
from jax.experimental import pallas as pl
from jax.experimental.pallas import tpu as pltpu
from jax.experimental.pallas import tpu_sc as plsc

N_NODES = 10000
N_EDGES = 320000
LATENT = 32
PACK = 4
LANES = PACK * LATENT
EPS = 1e-5

NC, NS = 2, 16
NW = NC * NS
E_W = N_EDGES // NW
K = 80
C = E_W // K
GRP = 5
NGRP = C // GRP

_mesh = plsc.VectorSubcoreMesh(
    core_axis_name="c", subcore_axis_name="s", num_cores=NC, num_subcores=NS)
_sc_params = pltpu.CompilerParams(use_tc_tiling_on_sc=False)


def _ln(z, bd1, s, b):
  mu = jnp.dot(z, bd1, preferred_element_type=jnp.float32) * (1.0 / LATENT)
  d = z - mu
  var = jnp.dot(d * d, bd1, preferred_element_type=jnp.float32) * (1.0 / LATENT)
  return d * lax.rsqrt(var + EPS) * s + b


def _enc_node_body(x_ref, w_ref, c_ref, s_ref, b_ref, one_ref, as_ref, ar_ref,
                   nl_ref, qs_ref, qr_ref):
  h = jnp.dot(x_ref[...], w_ref[...], preferred_element_type=jnp.float32)
  nl = _ln(h + c_ref[...], one_ref[...], s_ref[...], b_ref[...])
  nl_ref[...] = nl
  qs_ref[...] = jnp.dot(nl, as_ref[...], preferred_element_type=jnp.float32)
  qr_ref[...] = jnp.dot(nl, ar_ref[...], preferred_element_type=jnp.float32)


def _enc_edge_body(x_ref, w_ref, c_ref, s_ref, b_ref, one_ref, el_ref):
  y = jnp.dot(x_ref[...], w_ref[...], preferred_element_type=jnp.float32)
  h = y.reshape(y.shape[0] * 8, LANES)
  el_ref[...] = _ln(h + c_ref[...], one_ref[...], s_ref[...], b_ref[...])


def _edge_step_body(el_ref, g_ref, a_ref, c_ref, s_ref, b_ref,
                    one_ref, elo_ref, ne_ref):
  el = el_ref[...]
  z = (jnp.dot(el, a_ref[...], preferred_element_type=jnp.float32)
       + c_ref[...] + g_ref[...])
  ne = _ln(z, one_ref[...], s_ref[...], b_ref[...])
  elo_ref[...] = el + ne
  ne_ref[...] = ne


def _node_step_body(nl_ref, p0_ref, p1_ref, bn_ref, ba_ref, c_ref, s_ref,
                    b_ref, one_ref, as_ref, ar_ref, nlo_ref, qs_ref, qr_ref):
  nl = nl_ref[...]
  agg = p0_ref[...] + p1_ref[...]
  z = (jnp.dot(nl, bn_ref[...], preferred_element_type=jnp.float32)
       + jnp.dot(agg, ba_ref[...], preferred_element_type=jnp.float32)
       + c_ref[...])
  nl2 = nl + _ln(z, one_ref[...], s_ref[...], b_ref[...])
  nlo_ref[...] = nl2
  qs_ref[...] = jnp.dot(nl2, as_ref[...], preferred_element_type=jnp.float32)
  qr_ref[...] = jnp.dot(nl2, ar_ref[...], preferred_element_type=jnp.float32)


def _node_final_body(nl_ref, p0_ref, p1_ref, bn_ref, ba_ref, c_ref, s_ref,
                     b_ref, one_ref, ad_ref, cd_ref, out_ref):
  nl = nl_ref[...]
  agg = p0_ref[...] + p1_ref[...]
  z = (jnp.dot(nl, bn_ref[...], preferred_element_type=jnp.float32)
       + jnp.dot(agg, ba_ref[...], preferred_element_type=jnp.float32)
       + c_ref[...])
  nl2 = nl + _ln(z, one_ref[...], s_ref[...], b_ref[...])
  out_ref[...] = (jnp.dot(nl2, ad_ref[...], preferred_element_type=jnp.float32)
                  + cd_ref[...])


@functools.partial(
    pl.kernel,
    out_type=jax.ShapeDtypeStruct((N_EDGES, LATENT), jnp.float32),
    mesh=_mesh,
    scratch_types=(
        pltpu.VMEM((C, K), jnp.int32),
        pltpu.VMEM((C, K), jnp.int32),
        pltpu.VMEM((GRP * K, LATENT), jnp.float32),
        pltpu.VMEM((GRP * K, LATENT), jnp.float32),
        pltpu.VMEM((GRP * K, LATENT), jnp.float32),
        pltpu.VMEM((GRP * K, LATENT), jnp.float32),
        [pltpu.SemaphoreType.DMA for _ in range(2 * GRP)],
        [pltpu.SemaphoreType.DMA for _ in range(2 * GRP)],
        pltpu.SemaphoreType.DMA,
    ),
    compiler_params=_sc_params,
)
def _sc_gather(qs_hbm, qr_hbm, sidx_hbm, ridx_hbm, g_hbm,
               sidx_v, ridx_v, gbuf_a, rbuf_a, gbuf_b, rbuf_b,
               qsems, rsems, wsem):
  wid = lax.axis_index("s") * NC + lax.axis_index("c")
  pltpu.sync_copy(sidx_hbm.at[wid], sidx_v)
  pltpu.sync_copy(ridx_hbm.at[wid], ridx_v)

  def issue(sup, gbuf, rbuf, soff):
    qd, rd = [], []
    for b in range(GRP):
      j = sup * GRP + b
      qd.append(pltpu.async_copy(qs_hbm.at[sidx_v.at[j]],
                                 gbuf.at[pl.ds(b * K, K)], qsems[soff + b]))
      rd.append(pltpu.async_copy(qr_hbm.at[ridx_v.at[j]],
                                 rbuf.at[pl.ds(b * K, K)], rsems[soff + b]))
    return qd, rd

  def accum(gbuf, rbuf, qd, rd):
    for b in range(GRP):
      qd[b].wait()
      rd[b].wait()

      def row4(u, carry):
        base = b * K + u * 4
        for v in range(4):
          for h in (0, 16):
            gbuf[base + v, pl.ds(h, 16)] = (gbuf[base + v, pl.ds(h, 16)]
                                            + rbuf[base + v, pl.ds(h, 16)])
        return carry

      lax.fori_loop(0, K // 4, row4, 0)

  def write(sup, gbuf):
    row0 = wid * E_W + sup * GRP * K
    return pltpu.async_copy(gbuf.at[pl.ds(0, GRP * K)],
                            g_hbm.at[pl.ds(row0, GRP * K)], wsem)

  def pair(p, carry):
    sa, sb = 2 * p, 2 * p + 1
    qa, ra = issue(sa, gbuf_a, rbuf_a, 0)
    qb, rb = issue(sb, gbuf_b, rbuf_b, GRP)
    accum(gbuf_a, rbuf_a, qa, ra)
    wa = write(sa, gbuf_a)
    accum(gbuf_b, rbuf_b, qb, rb)
    wb = write(sb, gbuf_b)
    wa.wait()
    wb.wait()
    return carry

  lax.fori_loop(0, (C // GRP) // 2, pair, 0)
  qa, ra = issue((C // GRP) - 1, gbuf_a, rbuf_a, 0)
  accum(gbuf_a, rbuf_a, qa, ra)
  write((C // GRP) - 1, gbuf_a).wait()


@functools.partial(
    pl.kernel,
    out_type=jax.ShapeDtypeStruct((NC, N_NODES, LATENT), jnp.float32),
    mesh=_mesh,
    scratch_types=(
        pltpu.VMEM((C, K), jnp.int32),
        pltpu.VMEM((GRP * K, LATENT), jnp.float32),
        pltpu.VMEM((GRP * K, LATENT), jnp.float32),
        pltpu.VMEM_SHARED((N_NODES, LATENT), jnp.float32),
        pltpu.SemaphoreType.DMA,
        pltpu.SemaphoreType.DMA,
        pltpu.SemaphoreType.DMA,
        pltpu.SemaphoreType.DMA,
    ),
    compiler_params=_sc_params,
)
def _sc_scatter(ne_hbm, ridx_hbm, zero_hbm, p_hbm, ridx_v, nbuf_a, nbuf_b,
                shared, lsem_a, lsem_b, ssem_a, ssem_b):
  cid = lax.axis_index("c")
  sid = lax.axis_index("s")
  wid = sid * NC + cid
  pltpu.sync_copy(ridx_hbm.at[wid], ridx_v)

  @pl.when(sid == 0)
  def _():
    pltpu.sync_copy(zero_hbm, shared)

  plsc.subcore_barrier()

  def load(sup, nbuf, lsem):
    row0 = wid * E_W + sup * GRP * K
    return pltpu.async_copy(ne_hbm.at[pl.ds(row0, GRP * K)],
                            nbuf.at[pl.ds(0, GRP * K)], lsem)

  def scat(sup, nbuf, ssem):
    descs = []
    for b in range(GRP):
      j = sup * GRP + b
      descs.append(pltpu.async_copy(nbuf.at[pl.ds(b * K, K)],
                                    shared.at[ridx_v.at[j]], ssem, add=True))
    return descs

  def pair(p, carry):
    sa, sb = 2 * p, 2 * p + 1
    la = load(sa, nbuf_a, lsem_a)
    lb = load(sb, nbuf_b, lsem_b)
    la.wait()
    da = scat(sa, nbuf_a, ssem_a)
    lb.wait()
    db = scat(sb, nbuf_b, ssem_b)
    for d in da + db:
      d.wait()
    return carry

  lax.fori_loop(0, (C // GRP) // 2, pair, 0)
  tail = (C // GRP) - 1
  load(tail, nbuf_a, lsem_a).wait()
  for d in scat(tail, nbuf_a, ssem_a):
    d.wait()
  plsc.subcore_barrier()

  @pl.when(sid == 0)
  def _():
    pltpu.sync_copy(shared, p_hbm.at[cid])


@functools.partial(
    pl.kernel,
    out_type=jax.ShapeDtypeStruct((N_EDGES, 4), jnp.float32),
    mesh=_mesh,
    scratch_types=(pltpu.VMEM((E_W, 4), jnp.float32),),
    compiler_params=_sc_params,
)
def _sc_compact(ef_hbm, out_hbm, buf):
  wid = lax.axis_index("s") * NC + lax.axis_index("c")
  pltpu.sync_copy(ef_hbm.at[pl.ds(wid * E_W, E_W)], buf)
  pltpu.sync_copy(buf, out_hbm.at[pl.ds(wid * E_W, E_W)])


def _full(shape):
  return pl.BlockSpec(shape, lambda i: (0, 0))


def kernel(node_features, edge_features, senders, receivers,
           enc_node_W0, enc_node_b0, enc_node_W1, enc_node_b1,
           enc_node_ln_s, enc_node_ln_b,
           enc_edge_W0, enc_edge_b0, enc_edge_W1, enc_edge_b1,
           enc_edge_ln_s, enc_edge_ln_b,
           blk_edge_W0, blk_edge_b0, blk_edge_W1, blk_edge_b1,
           blk_edge_ln_s, blk_edge_ln_b,
           blk_node_W0, blk_node_b0, blk_node_W1, blk_node_b1,
           blk_node_ln_s, blk_node_ln_b,
           dec_W0, dec_b0, dec_W1, dec_b1):
  f32 = jnp.float32
  eye4 = jnp.eye(PACK, dtype=f32)
  bd = lambda a: jnp.kron(eye4, a)
  tile4 = lambda v: jnp.tile(v, PACK).reshape(1, -1)
  bd1 = bd(jnp.ones((LATENT, LATENT), f32))

  an = enc_node_W0 @ enc_node_W1
  cn = enc_node_b0 @ enc_node_W1 + enc_node_b1
  ae = enc_edge_W0 @ enc_edge_W1
  ce = enc_edge_b0 @ enc_edge_W1 + enc_edge_b1
  ea = jnp.einsum('tpq,tqr->tpr', blk_edge_W0, blk_edge_W1)
  ec = jnp.einsum('tq,tqr->tr', blk_edge_b0, blk_edge_W1) + blk_edge_b1
  na = jnp.einsum('tpq,tqr->tpr', blk_node_W0, blk_node_W1)
  nc = jnp.einsum('tq,tqr->tr', blk_node_b0, blk_node_W1) + blk_node_b1
  ad = dec_W0 @ dec_W1
  cd = dec_b0 @ dec_W1 + dec_b1

  nf4 = node_features.reshape(N_NODES // PACK, PACK * 128)
  s3d = senders.reshape(NW, C, K)
  r3d = receivers.reshape(NW, C, K)
  zeros_tab = jnp.zeros((N_NODES, LATENT), f32)

  nrows = N_NODES // PACK
  erows = N_EDGES // PACK
  eblk = 2000
  egrid = erows // eblk

  nl4, qs, qr = pl.pallas_call(
      _enc_node_body,
      out_shape=[jax.ShapeDtypeStruct((nrows, LANES), f32)] * 3,
  )(nf4, bd(an), tile4(cn), tile4(enc_node_ln_s), tile4(enc_node_ln_b),
    bd1, bd(ea[0, 0:32]), bd(ea[0, 32:64]))

  ef128 = _sc_compact(edge_features).reshape(N_EDGES // 32, 128)
  el4 = pl.pallas_call(
      _enc_edge_body,
      grid=(10,),
      in_specs=[pl.BlockSpec((N_EDGES // 320, 128), lambda i: (i, 0)),
                _full((128, 32 * LATENT)), _full((1, LANES)),
                _full((1, LANES)), _full((1, LANES)), _full((LANES, LANES))],
      out_specs=pl.BlockSpec((erows // 10, LANES), lambda i: (i, 0)),
      out_shape=jax.ShapeDtypeStruct((erows, LANES), f32),
  )(ef128, jnp.kron(jnp.eye(32, dtype=f32), ae), tile4(ce),
    tile4(enc_edge_ln_s), tile4(enc_edge_ln_b), bd1)

  for t in range(4):
    g = _sc_gather(qs.reshape(N_NODES, LATENT),
                   qr.reshape(N_NODES, LATENT), s3d, r3d)
    el4, ne = pl.pallas_call(
        _edge_step_body,
        grid=(egrid,),
        in_specs=[pl.BlockSpec((eblk, LANES), lambda i: (i, 0))] * 2 +
                 [_full((LANES, LANES)), _full((1, LANES)), _full((1, LANES)),
                  _full((1, LANES)), _full((LANES, LANES))],
        out_specs=[pl.BlockSpec((eblk, LANES), lambda i: (i, 0))] * 2,
        out_shape=[jax.ShapeDtypeStruct((erows, LANES), f32)] * 2,
    )(el4, g.reshape(erows, LANES),
      bd(ea[t, 64:96]), tile4(ec[t]), tile4(blk_edge_ln_s[t]),
      tile4(blk_edge_ln_b[t]), bd1)

    part = _sc_scatter(ne.reshape(N_EDGES, LATENT), r3d, zeros_tab)
    p0 = part[0].reshape(nrows, LANES)
    p1 = part[1].reshape(nrows, LANES)

    if t < 3:
      nl4, qs, qr = pl.pallas_call(
          _node_step_body,
          out_shape=[jax.ShapeDtypeStruct((nrows, LANES), f32)] * 3,
      )(nl4, p0, p1, bd(na[t, 0:32]), bd(na[t, 32:64]), tile4(nc[t]),
        tile4(blk_node_ln_s[t]), tile4(blk_node_ln_b[t]), bd1,
        bd(ea[t + 1, 0:32]), bd(ea[t + 1, 32:64]))
    else:
      out4 = pl.pallas_call(
          _node_final_body,
          out_shape=jax.ShapeDtypeStruct((nrows, PACK * 3), f32),
      )(nl4, p0, p1, bd(na[t, 0:32]), bd(na[t, 32:64]), tile4(nc[t]),
        tile4(blk_node_ln_s[t]), tile4(blk_node_ln_b[t]), bd1,
        bd(ad), tile4(cd))

  return out4.reshape(N_NODES, 3)


# speedup vs baseline: 1.2823x; 1.2823x over previous
import functools

import jax
import jax.numpy as jnp
from jax import lax
from jax.experimental import pallas as pl
from jax.experimental.pallas import tpu as pltpu
from jax.experimental.pallas import tpu_sc as plsc

N_NODES = 10000
N_EDGES = 320000
LATENT = 32
PACK = 4
LANES = PACK * LATENT
EPS = 1e-5

NC, NS = 2, 16
NW = NC * NS
E_W = N_EDGES // NW
K = 80
C = E_W // K
GRP = 5
NGRP = C // GRP

_mesh = plsc.VectorSubcoreMesh(
    core_axis_name="c", subcore_axis_name="s", num_cores=NC, num_subcores=NS)
_sc_params = pltpu.CompilerParams(use_tc_tiling_on_sc=False)


def _ln(z, bd1, s, b):
  mu = jnp.dot(z, bd1, preferred_element_type=jnp.float32) * (1.0 / LATENT)
  d = z - mu
  var = jnp.dot(d * d, bd1, preferred_element_type=jnp.float32) * (1.0 / LATENT)
  return d * lax.rsqrt(var + EPS) * s + b


def _enc_node_body(x_ref, w_ref, c_ref, s_ref, b_ref, one_ref, as_ref, ar_ref,
                   nl_ref, qs_ref, qr_ref):
  h = jnp.dot(x_ref[...], w_ref[...], preferred_element_type=jnp.float32)
  nl = _ln(h + c_ref[...], one_ref[...], s_ref[...], b_ref[...])
  nl_ref[...] = nl
  qs_ref[...] = jnp.dot(nl, as_ref[...], preferred_element_type=jnp.float32)
  qr_ref[...] = jnp.dot(nl, ar_ref[...], preferred_element_type=jnp.float32)


def _enc_edge_body(x_ref, w_ref, c_ref, s_ref, b_ref, one_ref, el_ref):
  y = jnp.dot(x_ref[...], w_ref[...], preferred_element_type=jnp.float32)
  y3 = y.reshape(y.shape[0] // PACK, PACK, LATENT)
  h = jnp.concatenate([y3[:, g, :] for g in range(PACK)], axis=1)
  el_ref[...] = _ln(h + c_ref[...], one_ref[...], s_ref[...], b_ref[...])


def _edge_step_body(el_ref, g_ref, a_ref, c_ref, s_ref, b_ref,
                    one_ref, elo_ref, ne_ref):
  el = el_ref[...]
  z = (jnp.dot(el, a_ref[...], preferred_element_type=jnp.float32)
       + c_ref[...] + g_ref[...])
  ne = _ln(z, one_ref[...], s_ref[...], b_ref[...])
  elo_ref[...] = el + ne
  ne_ref[...] = ne


def _node_step_body(nl_ref, p0_ref, p1_ref, p2_ref, p3_ref, bn_ref, ba_ref,
                    c_ref, s_ref, b_ref, one_ref, as_ref, ar_ref,
                    nlo_ref, qs_ref, qr_ref):
  nl = nl_ref[...]
  agg = (p0_ref[...] + p1_ref[...]) + (p2_ref[...] + p3_ref[...])
  z = (jnp.dot(nl, bn_ref[...], preferred_element_type=jnp.float32)
       + jnp.dot(agg, ba_ref[...], preferred_element_type=jnp.float32)
       + c_ref[...])
  nl2 = nl + _ln(z, one_ref[...], s_ref[...], b_ref[...])
  nlo_ref[...] = nl2
  qs_ref[...] = jnp.dot(nl2, as_ref[...], preferred_element_type=jnp.float32)
  qr_ref[...] = jnp.dot(nl2, ar_ref[...], preferred_element_type=jnp.float32)


def _node_final_body(nl_ref, p0_ref, p1_ref, p2_ref, p3_ref, bn_ref, ba_ref,
                     c_ref, s_ref, b_ref, one_ref, ad_ref, cd_ref, out_ref):
  nl = nl_ref[...]
  agg = (p0_ref[...] + p1_ref[...]) + (p2_ref[...] + p3_ref[...])
  z = (jnp.dot(nl, bn_ref[...], preferred_element_type=jnp.float32)
       + jnp.dot(agg, ba_ref[...], preferred_element_type=jnp.float32)
       + c_ref[...])
  nl2 = nl + _ln(z, one_ref[...], s_ref[...], b_ref[...])
  out_ref[...] = (jnp.dot(nl2, ad_ref[...], preferred_element_type=jnp.float32)
                  + cd_ref[...])


def _make_gather(n_edges):
  ew = n_edges // NW
  k = ew // C

  @functools.partial(
      pl.kernel,
      out_type=jax.ShapeDtypeStruct((n_edges, LATENT), jnp.float32),
      mesh=_mesh,
      scratch_types=(
          pltpu.VMEM((C, k), jnp.int32),
          pltpu.VMEM((C, k), jnp.int32),
          pltpu.VMEM((GRP * k, LATENT), jnp.float32),
          pltpu.VMEM((GRP * k, LATENT), jnp.float32),
          pltpu.VMEM((GRP * k, LATENT), jnp.float32),
          pltpu.VMEM((GRP * k, LATENT), jnp.float32),
          [pltpu.SemaphoreType.DMA for _ in range(2 * GRP)],
          [pltpu.SemaphoreType.DMA for _ in range(2 * GRP)],
          pltpu.SemaphoreType.DMA,
      ),
      compiler_params=_sc_params,
  )
  def gather(qs_hbm, qr_hbm, sidx_hbm, ridx_hbm, g_hbm,
             sidx_v, ridx_v, gbuf_a, rbuf_a, gbuf_b, rbuf_b,
             qsems, rsems, wsem):
    wid = lax.axis_index("s") * NC + lax.axis_index("c")
    pltpu.sync_copy(sidx_hbm.at[wid], sidx_v)
    pltpu.sync_copy(ridx_hbm.at[wid], ridx_v)

    def issue(sup, gbuf, rbuf, soff):
      qd, rd = [], []
      for b in range(GRP):
        j = sup * GRP + b
        qd.append(pltpu.async_copy(qs_hbm.at[sidx_v.at[j]],
                                   gbuf.at[pl.ds(b * k, k)], qsems[soff + b]))
        rd.append(pltpu.async_copy(qr_hbm.at[ridx_v.at[j]],
                                   rbuf.at[pl.ds(b * k, k)], rsems[soff + b]))
      return qd, rd

    def accum(gbuf, rbuf, qd, rd):
      for b in range(GRP):
        qd[b].wait()
        rd[b].wait()

        def row4(u, carry):
          base = b * k + u * 4
          for v in range(4):
            for h in (0, 16):
              gbuf[base + v, pl.ds(h, 16)] = (gbuf[base + v, pl.ds(h, 16)]
                                              + rbuf[base + v, pl.ds(h, 16)])
          return carry

        lax.fori_loop(0, k // 4, row4, 0)

    def write(sup, gbuf):
      row0 = wid * ew + sup * GRP * k
      return pltpu.async_copy(gbuf.at[pl.ds(0, GRP * k)],
                              g_hbm.at[pl.ds(row0, GRP * k)], wsem)

    def pair(p, carry):
      sa, sb = 2 * p, 2 * p + 1
      qa, ra = issue(sa, gbuf_a, rbuf_a, 0)
      qb, rb = issue(sb, gbuf_b, rbuf_b, GRP)
      accum(gbuf_a, rbuf_a, qa, ra)
      wa = write(sa, gbuf_a)
      accum(gbuf_b, rbuf_b, qb, rb)
      wb = write(sb, gbuf_b)
      wa.wait()
      wb.wait()
      return carry

    lax.fori_loop(0, (C // GRP) // 2, pair, 0)
    qa, ra = issue((C // GRP) - 1, gbuf_a, rbuf_a, 0)
    accum(gbuf_a, rbuf_a, qa, ra)
    write((C // GRP) - 1, gbuf_a).wait()

  return gather


def _make_scatter(n_edges):
  ew = n_edges // NW
  k = ew // C

  @functools.partial(
      pl.kernel,
      out_type=jax.ShapeDtypeStruct((NC, N_NODES, LATENT), jnp.float32),
      mesh=_mesh,
      scratch_types=(
          pltpu.VMEM((C, k), jnp.int32),
          pltpu.VMEM((GRP * k, LATENT), jnp.float32),
          pltpu.VMEM((GRP * k, LATENT), jnp.float32),
          pltpu.VMEM_SHARED((N_NODES, LATENT), jnp.float32),
          pltpu.SemaphoreType.DMA,
          pltpu.SemaphoreType.DMA,
          pltpu.SemaphoreType.DMA,
          pltpu.SemaphoreType.DMA,
      ),
      compiler_params=_sc_params,
  )
  def scatter(ne_hbm, ridx_hbm, zero_hbm, p_hbm, ridx_v, nbuf_a, nbuf_b,
              shared, lsem_a, lsem_b, ssem_a, ssem_b):
    cid = lax.axis_index("c")
    sid = lax.axis_index("s")
    wid = sid * NC + cid
    pltpu.sync_copy(ridx_hbm.at[wid], ridx_v)

    @pl.when(sid == 0)
    def _():
      pltpu.sync_copy(zero_hbm, shared)

    plsc.subcore_barrier()

    def load(sup, nbuf, lsem):
      row0 = wid * ew + sup * GRP * k
      return pltpu.async_copy(ne_hbm.at[pl.ds(row0, GRP * k)],
                              nbuf.at[pl.ds(0, GRP * k)], lsem)

    def scat(sup, nbuf, ssem):
      descs = []
      for b in range(GRP):
        j = sup * GRP + b
        descs.append(pltpu.async_copy(nbuf.at[pl.ds(b * k, k)],
                                      shared.at[ridx_v.at[j]], ssem, add=True))
      return descs

    def pair(p, carry):
      sa, sb = 2 * p, 2 * p + 1
      la = load(sa, nbuf_a, lsem_a)
      lb = load(sb, nbuf_b, lsem_b)
      la.wait()
      da = scat(sa, nbuf_a, ssem_a)
      lb.wait()
      db = scat(sb, nbuf_b, ssem_b)
      for d in da + db:
        d.wait()
      return carry

    lax.fori_loop(0, (C // GRP) // 2, pair, 0)
    tail = (C // GRP) - 1
    load(tail, nbuf_a, lsem_a).wait()
    for d in scat(tail, nbuf_a, ssem_a):
      d.wait()
    plsc.subcore_barrier()

    @pl.when(sid == 0)
    def _():
      pltpu.sync_copy(shared, p_hbm.at[cid])

  return scatter


_gather_half = _make_gather(N_EDGES // 2)
_scatter_half = _make_scatter(N_EDGES // 2)


def _full(shape):
  return pl.BlockSpec(shape, lambda i: (0, 0))


def kernel(node_features, edge_features, senders, receivers,
           enc_node_W0, enc_node_b0, enc_node_W1, enc_node_b1,
           enc_node_ln_s, enc_node_ln_b,
           enc_edge_W0, enc_edge_b0, enc_edge_W1, enc_edge_b1,
           enc_edge_ln_s, enc_edge_ln_b,
           blk_edge_W0, blk_edge_b0, blk_edge_W1, blk_edge_b1,
           blk_edge_ln_s, blk_edge_ln_b,
           blk_node_W0, blk_node_b0, blk_node_W1, blk_node_b1,
           blk_node_ln_s, blk_node_ln_b,
           dec_W0, dec_b0, dec_W1, dec_b1):
  f32 = jnp.float32
  eye4 = jnp.eye(PACK, dtype=f32)
  bd = lambda a: jnp.kron(eye4, a)
  tile4 = lambda v: jnp.tile(v, PACK).reshape(1, -1)
  bd1 = bd(jnp.ones((LATENT, LATENT), f32))

  an = enc_node_W0 @ enc_node_W1
  cn = enc_node_b0 @ enc_node_W1 + enc_node_b1
  ae = enc_edge_W0 @ enc_edge_W1
  ce = enc_edge_b0 @ enc_edge_W1 + enc_edge_b1
  ea = jnp.einsum('tpq,tqr->tpr', blk_edge_W0, blk_edge_W1)
  ec = jnp.einsum('tq,tqr->tr', blk_edge_b0, blk_edge_W1) + blk_edge_b1
  na = jnp.einsum('tpq,tqr->tpr', blk_node_W0, blk_node_W1)
  nc = jnp.einsum('tq,tqr->tr', blk_node_b0, blk_node_W1) + blk_node_b1
  ad = dec_W0 @ dec_W1
  cd = dec_b0 @ dec_W1 + dec_b1

  nf4 = node_features.reshape(N_NODES // PACK, PACK * 128)
  neh = N_EDGES // 2
  kh = neh // NW // C
  sh = [senders[h * neh:(h + 1) * neh].reshape(NW, C, kh) for h in (0, 1)]
  rh = [receivers[h * neh:(h + 1) * neh].reshape(NW, C, kh) for h in (0, 1)]
  zeros_tab = jnp.zeros((N_NODES, LATENT), f32)

  nrows = N_NODES // PACK
  erows = N_EDGES // PACK
  ehrows = erows // 2
  eblk = 2000
  egrid = ehrows // eblk

  nl4, qs, qr = pl.pallas_call(
      _enc_node_body,
      out_shape=[jax.ShapeDtypeStruct((nrows, LANES), f32)] * 3,
  )(nf4, bd(an), tile4(cn), tile4(enc_node_ln_s), tile4(enc_node_ln_b),
    bd1, bd(ea[0, 0:32]), bd(ea[0, 32:64]))

  el4 = [
      pl.pallas_call(
          _enc_edge_body,
          grid=(egrid,),
          in_specs=[pl.BlockSpec((PACK * eblk, 4),
                                 lambda i, h=h: (i + h * egrid, 0)),
                    _full((4, LATENT)), _full((1, LANES)),
                    _full((1, LANES)), _full((1, LANES)),
                    _full((LANES, LANES))],
          out_specs=pl.BlockSpec((eblk, LANES), lambda i: (i, 0)),
          out_shape=jax.ShapeDtypeStruct((ehrows, LANES), f32),
      )(edge_features, ae, tile4(ce), tile4(enc_edge_ln_s),
        tile4(enc_edge_ln_b), bd1)
      for h in (0, 1)
  ]

  for t in range(4):
    qs32 = qs.reshape(N_NODES, LATENT)
    qr32 = qr.reshape(N_NODES, LATENT)
    g = [_gather_half(qs32, qr32, sh[h], rh[h]) for h in (0, 1)]
    ne = [None, None]
    parts = [None, None]
    for h in (0, 1):
      el4[h], ne[h] = pl.pallas_call(
          _edge_step_body,
          grid=(egrid,),
          in_specs=[pl.BlockSpec((eblk, LANES), lambda i: (i, 0))] * 2 +
                   [_full((LANES, LANES)), _full((1, LANES)),
                    _full((1, LANES)), _full((1, LANES)),
                    _full((LANES, LANES))],
          out_specs=[pl.BlockSpec((eblk, LANES), lambda i: (i, 0))] * 2,
          out_shape=[jax.ShapeDtypeStruct((ehrows, LANES), f32)] * 2,
      )(el4[h], g[h].reshape(ehrows, LANES),
        bd(ea[t, 64:96]), tile4(ec[t]), tile4(blk_edge_ln_s[t]),
        tile4(blk_edge_ln_b[t]), bd1)
      parts[h] = _scatter_half(ne[h].reshape(neh, LATENT), rh[h], zeros_tab)
    ps = [parts[h][c].reshape(nrows, LANES) for h in (0, 1) for c in (0, 1)]

    if t < 3:
      nl4, qs, qr = pl.pallas_call(
          _node_step_body,
          out_shape=[jax.ShapeDtypeStruct((nrows, LANES), f32)] * 3,
      )(nl4, *ps, bd(na[t, 0:32]), bd(na[t, 32:64]), tile4(nc[t]),
        tile4(blk_node_ln_s[t]), tile4(blk_node_ln_b[t]), bd1,
        bd(ea[t + 1, 0:32]), bd(ea[t + 1, 32:64]))
    else:
      out4 = pl.pallas_call(
          _node_final_body,
          out_shape=jax.ShapeDtypeStruct((nrows, PACK * 3), f32),
      )(nl4, *ps, bd(na[t, 0:32]), bd(na[t, 32:64]), tile4(nc[t]),
        tile4(blk_node_ln_s[t]), tile4(blk_node_ln_b[t]), bd1,
        bd(ad), tile4(cd))

  return out4.reshape(N_NODES, 3)


# speedup vs baseline: 1.3360x; 1.0419x over previous
import functools

import jax
import jax.numpy as jnp
from jax import lax
from jax.experimental import pallas as pl
from jax.experimental.pallas import tpu as pltpu
from jax.experimental.pallas import tpu_sc as plsc

N_NODES = 10000
N_EDGES = 320000
LATENT = 32
PACK = 4
LANES = PACK * LATENT
EPS = 1e-5

NC, NS = 2, 16
NW = NC * NS
E_W = N_EDGES // NW
K = 80
C = E_W // K
GRP = 5
NGRP = C // GRP

_mesh = plsc.VectorSubcoreMesh(
    core_axis_name="c", subcore_axis_name="s", num_cores=NC, num_subcores=NS)
_sc_params = pltpu.CompilerParams(use_tc_tiling_on_sc=False)


def _ln(z, bd1, s, b):
  mu = jnp.dot(z, bd1, preferred_element_type=jnp.float32) * (1.0 / LATENT)
  d = z - mu
  var = jnp.dot(d * d, bd1, preferred_element_type=jnp.float32) * (1.0 / LATENT)
  return d * lax.rsqrt(var + EPS) * s + b


def _enc_node_body(x_ref, w_ref, c_ref, s_ref, b_ref, one_ref, as_ref, ar_ref,
                   nl_ref, qs_ref, qr_ref):
  h = jnp.dot(x_ref[...], w_ref[...], preferred_element_type=jnp.float32)
  nl = _ln(h + c_ref[...], one_ref[...], s_ref[...], b_ref[...])
  nl_ref[...] = nl
  qs_ref[...] = jnp.dot(nl, as_ref[...], preferred_element_type=jnp.float32)
  qr_ref[...] = jnp.dot(nl, ar_ref[...], preferred_element_type=jnp.float32)


def _enc_edge_body(x_ref, w_ref, c_ref, s_ref, b_ref, one_ref, el_ref):
  y = jnp.dot(x_ref[...], w_ref[...], preferred_element_type=jnp.float32)
  y3 = y.reshape(y.shape[0] // PACK, PACK, LATENT)
  h = jnp.concatenate([y3[:, g, :] for g in range(PACK)], axis=1)
  el_ref[...] = _ln(h + c_ref[...], one_ref[...], s_ref[...], b_ref[...])


def _edge_step_body(el_ref, g_ref, a_ref, c_ref, s_ref, b_ref,
                    one_ref, elo_ref, ne_ref):
  el = el_ref[...]
  z = (jnp.dot(el, a_ref[...], preferred_element_type=jnp.float32)
       + c_ref[...] + g_ref[...])
  ne = _ln(z, one_ref[...], s_ref[...], b_ref[...])
  elo_ref[...] = el + ne
  ne_ref[...] = ne


def _node_step_body(nl_ref, p0_ref, p1_ref, bn_ref, ba_ref, c_ref, s_ref,
                    b_ref, one_ref, as_ref, ar_ref, nlo_ref, qs_ref, qr_ref):
  nl = nl_ref[...]
  agg = p0_ref[...] + p1_ref[...]
  z = (jnp.dot(nl, bn_ref[...], preferred_element_type=jnp.float32)
       + jnp.dot(agg, ba_ref[...], preferred_element_type=jnp.float32)
       + c_ref[...])
  nl2 = nl + _ln(z, one_ref[...], s_ref[...], b_ref[...])
  nlo_ref[...] = nl2
  qs_ref[...] = jnp.dot(nl2, as_ref[...], preferred_element_type=jnp.float32)
  qr_ref[...] = jnp.dot(nl2, ar_ref[...], preferred_element_type=jnp.float32)


def _node_final_body(nl_ref, p0_ref, p1_ref, bn_ref, ba_ref, c_ref, s_ref,
                     b_ref, one_ref, ad_ref, cd_ref, out_ref):
  nl = nl_ref[...]
  agg = p0_ref[...] + p1_ref[...]
  z = (jnp.dot(nl, bn_ref[...], preferred_element_type=jnp.float32)
       + jnp.dot(agg, ba_ref[...], preferred_element_type=jnp.float32)
       + c_ref[...])
  nl2 = nl + _ln(z, one_ref[...], s_ref[...], b_ref[...])
  out_ref[...] = (jnp.dot(nl2, ad_ref[...], preferred_element_type=jnp.float32)
                  + cd_ref[...])


def _make_gather(n_edges):
  ew = n_edges // NW
  k = ew // C

  @functools.partial(
      pl.kernel,
      out_type=jax.ShapeDtypeStruct((n_edges, LATENT), jnp.float32),
      mesh=_mesh,
      scratch_types=(
          pltpu.VMEM((C, k), jnp.int32),
          pltpu.VMEM((C, k), jnp.int32),
          pltpu.VMEM((GRP * k, LATENT), jnp.float32),
          pltpu.VMEM((GRP * k, LATENT), jnp.float32),
          pltpu.VMEM((GRP * k, LATENT), jnp.float32),
          pltpu.VMEM((GRP * k, LATENT), jnp.float32),
          [pltpu.SemaphoreType.DMA for _ in range(2 * GRP)],
          [pltpu.SemaphoreType.DMA for _ in range(2 * GRP)],
          pltpu.SemaphoreType.DMA,
      ),
      compiler_params=_sc_params,
  )
  def gather(qs_hbm, qr_hbm, sidx_hbm, ridx_hbm, g_hbm,
             sidx_v, ridx_v, gbuf_a, rbuf_a, gbuf_b, rbuf_b,
             qsems, rsems, wsem):
    wid = lax.axis_index("s") * NC + lax.axis_index("c")
    pltpu.sync_copy(sidx_hbm.at[wid], sidx_v)
    pltpu.sync_copy(ridx_hbm.at[wid], ridx_v)

    def issue(sup, gbuf, rbuf, soff):
      qd, rd = [], []
      for b in range(GRP):
        j = sup * GRP + b
        qd.append(pltpu.async_copy(qs_hbm.at[sidx_v.at[j]],
                                   gbuf.at[pl.ds(b * k, k)], qsems[soff + b]))
        rd.append(pltpu.async_copy(qr_hbm.at[ridx_v.at[j]],
                                   rbuf.at[pl.ds(b * k, k)], rsems[soff + b]))
      return qd, rd

    def accum(gbuf, rbuf, qd, rd):
      for b in range(GRP):
        qd[b].wait()
        rd[b].wait()

        def row4(u, carry):
          base = b * k + u * 4
          for v in range(4):
            for h in (0, 16):
              gbuf[base + v, pl.ds(h, 16)] = (gbuf[base + v, pl.ds(h, 16)]
                                              + rbuf[base + v, pl.ds(h, 16)])
          return carry

        lax.fori_loop(0, k // 4, row4, 0)

    def write(sup, gbuf):
      row0 = wid * ew + sup * GRP * k
      return pltpu.async_copy(gbuf.at[pl.ds(0, GRP * k)],
                              g_hbm.at[pl.ds(row0, GRP * k)], wsem)

    def pair(p, carry):
      sa, sb = 2 * p, 2 * p + 1
      qa, ra = issue(sa, gbuf_a, rbuf_a, 0)
      qb, rb = issue(sb, gbuf_b, rbuf_b, GRP)
      accum(gbuf_a, rbuf_a, qa, ra)
      wa = write(sa, gbuf_a)
      accum(gbuf_b, rbuf_b, qb, rb)
      wb = write(sb, gbuf_b)
      wa.wait()
      wb.wait()
      return carry

    lax.fori_loop(0, (C // GRP) // 2, pair, 0)
    qa, ra = issue((C // GRP) - 1, gbuf_a, rbuf_a, 0)
    accum(gbuf_a, rbuf_a, qa, ra)
    write((C // GRP) - 1, gbuf_a).wait()

  return gather


def _make_scatter(n_edges):
  ew = n_edges // NW
  k = ew // C

  @functools.partial(
      pl.kernel,
      out_type=jax.ShapeDtypeStruct((NC, N_NODES, LATENT), jnp.float32),
      mesh=_mesh,
      scratch_types=(
          pltpu.VMEM((C, k), jnp.int32),
          pltpu.VMEM((GRP * k, LATENT), jnp.float32),
          pltpu.VMEM((GRP * k, LATENT), jnp.float32),
          pltpu.VMEM_SHARED((N_NODES, LATENT), jnp.float32),
          pltpu.SemaphoreType.DMA,
          pltpu.SemaphoreType.DMA,
          pltpu.SemaphoreType.DMA,
          pltpu.SemaphoreType.DMA,
      ),
      compiler_params=_sc_params,
  )
  def scatter(ne_hbm, ridx_hbm, zero_hbm, p_hbm, ridx_v, nbuf_a, nbuf_b,
              shared, lsem_a, lsem_b, ssem_a, ssem_b):
    cid = lax.axis_index("c")
    sid = lax.axis_index("s")
    wid = sid * NC + cid
    pltpu.sync_copy(ridx_hbm.at[wid], ridx_v)

    @pl.when(sid == 0)
    def _():
      pltpu.sync_copy(zero_hbm, shared)

    plsc.subcore_barrier()

    def load(sup, nbuf, lsem):
      row0 = wid * ew + sup * GRP * k
      return pltpu.async_copy(ne_hbm.at[pl.ds(row0, GRP * k)],
                              nbuf.at[pl.ds(0, GRP * k)], lsem)

    def scat(sup, nbuf, ssem):
      descs = []
      for b in range(GRP):
        j = sup * GRP + b
        descs.append(pltpu.async_copy(nbuf.at[pl.ds(b * k, k)],
                                      shared.at[ridx_v.at[j]], ssem, add=True))
      return descs

    def pair(p, carry):
      sa, sb = 2 * p, 2 * p + 1
      la = load(sa, nbuf_a, lsem_a)
      lb = load(sb, nbuf_b, lsem_b)
      la.wait()
      da = scat(sa, nbuf_a, ssem_a)
      lb.wait()
      db = scat(sb, nbuf_b, ssem_b)
      for d in da + db:
        d.wait()
      return carry

    lax.fori_loop(0, (C // GRP) // 2, pair, 0)
    tail = (C // GRP) - 1
    load(tail, nbuf_a, lsem_a).wait()
    for d in scat(tail, nbuf_a, ssem_a):
      d.wait()
    plsc.subcore_barrier()

    @pl.when(sid == 0)
    def _():
      pltpu.sync_copy(shared, p_hbm.at[cid])

  return scatter


_sc_gather = _make_gather(N_EDGES)
_sc_scatter = _make_scatter(N_EDGES)


def _full(shape):
  return pl.BlockSpec(shape, lambda i: (0, 0))


def kernel(node_features, edge_features, senders, receivers,
           enc_node_W0, enc_node_b0, enc_node_W1, enc_node_b1,
           enc_node_ln_s, enc_node_ln_b,
           enc_edge_W0, enc_edge_b0, enc_edge_W1, enc_edge_b1,
           enc_edge_ln_s, enc_edge_ln_b,
           blk_edge_W0, blk_edge_b0, blk_edge_W1, blk_edge_b1,
           blk_edge_ln_s, blk_edge_ln_b,
           blk_node_W0, blk_node_b0, blk_node_W1, blk_node_b1,
           blk_node_ln_s, blk_node_ln_b,
           dec_W0, dec_b0, dec_W1, dec_b1):
  f32 = jnp.float32
  eye4 = jnp.eye(PACK, dtype=f32)
  bd = lambda a: jnp.kron(eye4, a)
  tile4 = lambda v: jnp.tile(v, PACK).reshape(1, -1)
  bd1 = bd(jnp.ones((LATENT, LATENT), f32))

  an = enc_node_W0 @ enc_node_W1
  cn = enc_node_b0 @ enc_node_W1 + enc_node_b1
  ae = enc_edge_W0 @ enc_edge_W1
  ce = enc_edge_b0 @ enc_edge_W1 + enc_edge_b1
  ea = jnp.einsum('tpq,tqr->tpr', blk_edge_W0, blk_edge_W1)
  ec = jnp.einsum('tq,tqr->tr', blk_edge_b0, blk_edge_W1) + blk_edge_b1
  na = jnp.einsum('tpq,tqr->tpr', blk_node_W0, blk_node_W1)
  nc = jnp.einsum('tq,tqr->tr', blk_node_b0, blk_node_W1) + blk_node_b1
  ad = dec_W0 @ dec_W1
  cd = dec_b0 @ dec_W1 + dec_b1

  nf4 = node_features.reshape(N_NODES // PACK, PACK * 128)
  s3d = senders.reshape(NW, C, K)
  r3d = receivers.reshape(NW, C, K)
  zeros_tab = jnp.zeros((N_NODES, LATENT), f32)

  nrows = N_NODES // PACK
  erows = N_EDGES // PACK
  eblk = 2000
  egrid = erows // eblk

  nl4, qs, qr = pl.pallas_call(
      _enc_node_body,
      out_shape=[jax.ShapeDtypeStruct((nrows, LANES), f32)] * 3,
  )(nf4, bd(an), tile4(cn), tile4(enc_node_ln_s), tile4(enc_node_ln_b),
    bd1, bd(ea[0, 0:32]), bd(ea[0, 32:64]))

  el4 = pl.pallas_call(
      _enc_edge_body,
      grid=(egrid,),
      in_specs=[pl.BlockSpec((PACK * eblk, 4), lambda i: (i, 0)),
                _full((4, LATENT)), _full((1, LANES)),
                _full((1, LANES)), _full((1, LANES)), _full((LANES, LANES))],
      out_specs=pl.BlockSpec((eblk, LANES), lambda i: (i, 0)),
      out_shape=jax.ShapeDtypeStruct((erows, LANES), f32),
  )(edge_features, ae, tile4(ce), tile4(enc_edge_ln_s), tile4(enc_edge_ln_b),
    bd1)

  for t in range(4):
    g = _sc_gather(qs.reshape(N_NODES, LATENT),
                   qr.reshape(N_NODES, LATENT), s3d, r3d)
    el4, ne = pl.pallas_call(
        _edge_step_body,
        grid=(egrid,),
        in_specs=[pl.BlockSpec((eblk, LANES), lambda i: (i, 0))] * 2 +
                 [_full((LANES, LANES)), _full((1, LANES)), _full((1, LANES)),
                  _full((1, LANES)), _full((LANES, LANES))],
        out_specs=[pl.BlockSpec((eblk, LANES), lambda i: (i, 0))] * 2,
        out_shape=[jax.ShapeDtypeStruct((erows, LANES), f32)] * 2,
    )(el4, g.reshape(erows, LANES),
      bd(ea[t, 64:96]), tile4(ec[t]), tile4(blk_edge_ln_s[t]),
      tile4(blk_edge_ln_b[t]), bd1)

    part = _sc_scatter(ne.reshape(N_EDGES, LATENT), r3d, zeros_tab)
    p0 = part[0].reshape(nrows, LANES)
    p1 = part[1].reshape(nrows, LANES)

    if t < 3:
      nl4, qs, qr = pl.pallas_call(
          _node_step_body,
          out_shape=[jax.ShapeDtypeStruct((nrows, LANES), f32)] * 3,
      )(nl4, p0, p1, bd(na[t, 0:32]), bd(na[t, 32:64]), tile4(nc[t]),
        tile4(blk_node_ln_s[t]), tile4(blk_node_ln_b[t]), bd1,
        bd(ea[t + 1, 0:32]), bd(ea[t + 1, 32:64]))
    else:
      out4 = pl.pallas_call(
          _node_final_body,
          out_shape=jax.ShapeDtypeStruct((nrows, PACK * 3), f32),
      )(nl4, p0, p1, bd(na[t, 0:32]), bd(na[t, 32:64]), tile4(nc[t]),
        tile4(blk_node_ln_s[t]), tile4(blk_node_ln_b[t]), bd1,
        bd(ad), tile4(cd))

  return out4.reshape(N_NODES, 3)
